# Initial kernel scaffold; baseline (speedup 1.0000x reference)
import jax
import jax.numpy as jnp
from jax.experimental import pallas as pl


def kernel(x, edge_index, W1, b1, W2, b2, Wfc, bfc):
    raise NotImplementedError("write your pallas kernel here")


# speedup vs baseline: 15.6422x; 15.6422x over previous
import functools

import jax
import jax.numpy as jnp
from jax import lax
from jax.experimental import pallas as pl
from jax.experimental.pallas import tpu as pltpu
from jax.experimental.pallas import tpu_sc as plsc

N_NODES = 10000
N_PAD = 10240
D_HID = 64
NC = 2
NS = 16
NW = NC * NS
CHUNK = 128
ROWS_PER_W = N_PAD // NS
ZROWS = 64

_mesh = plsc.VectorSubcoreMesh(core_axis_name="c", subcore_axis_name="s")


def _make_deg_kernel(e_pad):
    epw = e_pad // NW
    nit = epw // CHUNK

    @functools.partial(
        pl.kernel,
        out_type=jax.ShapeDtypeStruct((NC, N_PAD), jnp.float32),
        mesh=_mesh,
        scratch_types=[
            pltpu.VMEM((CHUNK,), jnp.int32),
            pltpu.VMEM((CHUNK,), jnp.float32),
            pltpu.VMEM_SHARED((N_PAD,), jnp.float32),
        ],
    )
    def deg_k(dst_hbm, zeros_hbm, ones_hbm, out_hbm, didx_v, ones_v, acc_s):
        cid = lax.axis_index("c")
        sid = lax.axis_index("s")
        wid = sid * NC + cid
        row0 = sid * ROWS_PER_W
        pltpu.sync_copy(ones_hbm, ones_v)
        pltpu.sync_copy(zeros_hbm, acc_s.at[pl.ds(row0, ROWS_PER_W)])
        plsc.subcore_barrier()
        base0 = wid * epw

        def body(i, carry):
            pltpu.sync_copy(dst_hbm.at[pl.ds(base0 + i * CHUNK, CHUNK)], didx_v)
            pltpu.sync_copy(ones_v, acc_s.at[didx_v], add=True)
            return carry

        lax.fori_loop(0, nit, body, 0)
        plsc.subcore_barrier()
        pltpu.sync_copy(acc_s.at[pl.ds(row0, ROWS_PER_W)],
                        out_hbm.at[cid, pl.ds(row0, ROWS_PER_W)])

    return deg_k


def _make_mp_kernel(e_pad):
    epw = e_pad // NW
    nit = epw // CHUNK

    @functools.partial(
        pl.kernel,
        out_type=jax.ShapeDtypeStruct((NC, N_PAD, D_HID), jnp.float32),
        mesh=_mesh,
        compiler_params=pltpu.CompilerParams(use_tc_tiling_on_sc=False),
        scratch_types=[
            pltpu.VMEM((CHUNK,), jnp.int32),
            pltpu.VMEM((CHUNK,), jnp.int32),
            pltpu.VMEM((CHUNK, D_HID), jnp.float32),
            pltpu.VMEM_SHARED((N_PAD, D_HID), jnp.float32),
            pltpu.SemaphoreType.DMA,
        ],
    )
    def mp_k(g_hbm, src_hbm, dst_hbm, zeros_hbm, out_hbm,
             sidx_v, didx_v, rows_v, acc_s, sem):
        cid = lax.axis_index("c")
        sid = lax.axis_index("s")
        wid = sid * NC + cid
        row0 = sid * ROWS_PER_W

        def zbody(j, carry):
            pltpu.sync_copy(zeros_hbm, acc_s.at[pl.ds(row0 + j * ZROWS, ZROWS)])
            return carry

        lax.fori_loop(0, ROWS_PER_W // ZROWS, zbody, 0)
        plsc.subcore_barrier()
        base0 = wid * epw

        def body(i, carry):
            b = base0 + i * CHUNK
            pltpu.sync_copy(src_hbm.at[pl.ds(b, CHUNK)], sidx_v)
            pltpu.sync_copy(dst_hbm.at[pl.ds(b, CHUNK)], didx_v)
            pltpu.async_copy(g_hbm.at[sidx_v], rows_v, sem).wait()
            pltpu.sync_copy(rows_v, acc_s.at[didx_v], add=True)
            return carry

        lax.fori_loop(0, nit, body, 0)
        plsc.subcore_barrier()
        pltpu.sync_copy(acc_s.at[pl.ds(row0, ROWS_PER_W)],
                        out_hbm.at[cid, pl.ds(row0, ROWS_PER_W)])

    return mp_k


def _tc1_body(degp_ref, x_ref, w1_ref, ones_ref, dinvd_ref, g1_ref):
    deg = degp_ref[0:1, :] + degp_ref[1:2, :]
    dinv = jnp.where(deg > 0.0, lax.rsqrt(deg), 0.0)
    dmat = lax.dot_general(dinv, ones_ref[...], (((0,), (0,)), ((), ())),
                           preferred_element_type=jnp.float32)
    dinvd_ref[...] = dmat
    g1_ref[...] = jnp.dot(x_ref[...], w1_ref[...],
                          preferred_element_type=jnp.float32) * dmat


def _tc2_body(sp_ref, dinvd_ref, b_ref, w2_ref, g2_ref):
    s = sp_ref[0] + sp_ref[1]
    dmat = dinvd_ref[...]
    h = jnp.maximum(s * dmat + b_ref[...], 0.0)
    g2_ref[...] = jnp.dot(h, w2_ref[...],
                          preferred_element_type=jnp.float32) * dmat


def _tc3_body(sp_ref, dinvd_ref, b_ref, wfc_ref, bfc_ref, q_ref):
    s = sp_ref[0] + sp_ref[1]
    h = jnp.maximum(s * dinvd_ref[...] + b_ref[...], 0.0)
    q_ref[...] = jnp.dot(h, wfc_ref[...],
                         preferred_element_type=jnp.float32) + bfc_ref[...]


def kernel(x, edge_index, W1, b1, W2, b2, Wfc, bfc):
    n = x.shape[0]
    e = edge_index.shape[1]
    d_in = x.shape[1]
    n_cls = Wfc.shape[1]
    e_tot = e + n
    e_pad = -(-e_tot // (NW * CHUNK)) * (NW * CHUNK)

    src = edge_index[0].astype(jnp.int32)
    dst = edge_index[1].astype(jnp.int32)
    loop = jnp.arange(n, dtype=jnp.int32)
    padv = jnp.full((e_pad - e_tot,), n, dtype=jnp.int32)
    src_f = jnp.concatenate([src, loop, padv])
    dst_f = jnp.concatenate([dst, loop, padv])
    x_pad = jnp.pad(x, ((0, N_PAD - n), (0, 0)))

    zeros1d = jnp.zeros((ROWS_PER_W,), jnp.float32)
    ones1d = jnp.ones((CHUNK,), jnp.float32)
    zeros2d = jnp.zeros((ZROWS, D_HID), jnp.float32)
    ones_row = jnp.ones((1, D_HID), jnp.float32)

    deg_k = _make_deg_kernel(e_pad)
    mp_k = _make_mp_kernel(e_pad)

    degp = deg_k(dst_f, zeros1d, ones1d)

    dinv_d, g1 = pl.pallas_call(
        _tc1_body,
        out_shape=[
            jax.ShapeDtypeStruct((N_PAD, D_HID), jnp.float32),
            jax.ShapeDtypeStruct((N_PAD, D_HID), jnp.float32),
        ],
    )(degp, x_pad, W1, ones_row)

    s1p = mp_k(g1, src_f, dst_f, zeros2d)

    g2 = pl.pallas_call(
        _tc2_body,
        out_shape=jax.ShapeDtypeStruct((N_PAD, D_HID), jnp.float32),
    )(s1p, dinv_d, b1.reshape(1, D_HID), W2)

    s2p = mp_k(g2, src_f, dst_f, zeros2d)

    q = pl.pallas_call(
        _tc3_body,
        out_shape=jax.ShapeDtypeStruct((N_PAD, n_cls), jnp.float32),
    )(s2p, dinv_d, b2.reshape(1, D_HID), Wfc, bfc.reshape(1, n_cls))

    return q[:n]
